# baseline (speedup 1.0000x reference)
import jax
import jax.numpy as jnp
from jax import lax
from jax.experimental import pallas as pl
from jax.experimental.pallas import tpu as pltpu

B = 256
NM = 1024
FEAT = 12288
KBLK = 1536
KSTEPS = FEAT // KBLK
FBLK = 1024
FSTEPS = FEAT // FBLK
GRID_N = 32


def _argmin_kernel(xt_ref, wt_ref, idx_ref, acc_ref):
    k = pl.program_id(0)
    part = lax.dot_general(
        xt_ref[...], wt_ref[...],
        dimension_numbers=(((0,), (0,)), ((), ())),
        preferred_element_type=jnp.float32,
    )
    w2 = jnp.zeros((1, NM), jnp.float32)
    for c in range(KBLK // 128):
        blk = wt_ref[c * 128:(c + 1) * 128, :]
        w2 = w2 + jnp.sum(blk * blk, axis=0, keepdims=True)
    upd = w2 - 2.0 * part

    @pl.when(k == 0)
    def _init():
        acc_ref[...] = upd

    @pl.when(k > 0)
    def _acc():
        acc_ref[...] += upd

    @pl.when(k == KSTEPS - 1)
    def _finish():
        scores = acc_ref[...]
        minv = jnp.min(scores, axis=1, keepdims=True)
        iota = lax.broadcasted_iota(jnp.int32, scores.shape, 1)
        idx = jnp.min(jnp.where(scores == minv, iota, NM), axis=1,
                      keepdims=True)
        flat = (idx % GRID_N) * GRID_N + idx // GRID_N
        idx_ref[...] = flat


def _compute_indices(xt, wt):
    return pl.pallas_call(
        _argmin_kernel,
        grid=(KSTEPS,),
        in_specs=[
            pl.BlockSpec((KBLK, B), lambda k: (k, 0)),
            pl.BlockSpec((KBLK, NM), lambda k: (k, 0)),
        ],
        out_specs=pl.BlockSpec((B, 1), lambda k: (0, 0)),
        out_shape=jax.ShapeDtypeStruct((B, 1), jnp.int32),
        scratch_shapes=[pltpu.VMEM((B, NM), jnp.float32)],
    )(xt, wt)


def _gather_kernel(wt_ref, idx_ref, out_ref):
    onehot = jnp.where(
        lax.broadcasted_iota(jnp.int32, (NM, B), 0) == idx_ref[...],
        1.0, 0.0).astype(jnp.float32)
    out_ref[...] = lax.dot_general(
        wt_ref[...], onehot,
        dimension_numbers=(((1,), (0,)), ((), ())),
        preferred_element_type=jnp.float32,
        precision=lax.Precision.HIGHEST,
    )


def _gather_rows(wt, idx_row):
    return pl.pallas_call(
        _gather_kernel,
        grid=(FSTEPS,),
        in_specs=[
            pl.BlockSpec((FBLK, NM), lambda k: (k, 0)),
            pl.BlockSpec((1, B), lambda k: (0, 0)),
        ],
        out_specs=pl.BlockSpec((FBLK, B), lambda k: (k, 0)),
        out_shape=jax.ShapeDtypeStruct((FEAT, B), jnp.float32),
    )(wt, idx_row)


def kernel(x, weights):
    xt = jnp.transpose(x, (1, 3, 2, 0)).reshape(FEAT, B)
    wt = jnp.transpose(weights, (2, 4, 3, 0, 1)).reshape(FEAT, NM)
    idx = _compute_indices(xt, wt)
    out_fm = _gather_rows(wt, idx.reshape(1, B))
    out4 = out_fm.reshape(64, 3, 64, B)
    return jnp.transpose(out4, (3, 0, 2, 1))


# speedup vs baseline: 1.3180x; 1.3180x over previous
import jax
import jax.numpy as jnp
from jax import lax
from jax.experimental import pallas as pl
from jax.experimental.pallas import tpu as pltpu

B = 256
NM = 1024
FEAT = 12288
KBLK = 1536
KSTEPS = FEAT // KBLK
FBLK = 1024
FSTEPS = FEAT // FBLK
GRID_N = 32


def _argmin_kernel(xt_ref, wt_ref, idx_ref, acc_ref):
    k = pl.program_id(0)
    part = lax.dot_general(
        xt_ref[...], wt_ref[...],
        dimension_numbers=(((0,), (0,)), ((), ())),
        preferred_element_type=jnp.float32,
    )
    w2 = jnp.zeros((1, NM), jnp.float32)
    for c in range(KBLK // 128):
        blk = wt_ref[c * 128:(c + 1) * 128, :]
        w2 = w2 + jnp.sum(blk * blk, axis=0, keepdims=True)
    upd = w2 - 2.0 * part

    @pl.when(k == 0)
    def _init():
        acc_ref[...] = upd

    @pl.when(k > 0)
    def _acc():
        acc_ref[...] += upd

    @pl.when(k == KSTEPS - 1)
    def _finish():
        scores = acc_ref[...]
        minv = jnp.min(scores, axis=1, keepdims=True)
        iota = lax.broadcasted_iota(jnp.int32, scores.shape, 1)
        idx = jnp.min(jnp.where(scores == minv, iota, NM), axis=1,
                      keepdims=True)
        flat = (idx % GRID_N) * GRID_N + idx // GRID_N
        idx_ref[...] = flat


def _compute_indices(xt, wt):
    return pl.pallas_call(
        _argmin_kernel,
        grid=(KSTEPS,),
        in_specs=[
            pl.BlockSpec((KBLK, B), lambda k: (k, 0)),
            pl.BlockSpec((KBLK, NM), lambda k: (k, 0)),
        ],
        out_specs=pl.BlockSpec((B, 1), lambda k: (0, 0)),
        out_shape=jax.ShapeDtypeStruct((B, 1), jnp.int32),
        scratch_shapes=[pltpu.VMEM((B, NM), jnp.float32)],
    )(xt, wt)


def _gather_kernel(wt_ref, idx_ref, out_ref):
    onehot = jnp.where(
        lax.broadcasted_iota(jnp.int32, (NM, B), 0) == idx_ref[...],
        1.0, 0.0).astype(jnp.float32)
    out_ref[...] = lax.dot_general(
        wt_ref[...], onehot,
        dimension_numbers=(((1,), (0,)), ((), ())),
        preferred_element_type=jnp.float32,
    )


def _gather_rows(wt, idx_row):
    return pl.pallas_call(
        _gather_kernel,
        grid=(FSTEPS,),
        in_specs=[
            pl.BlockSpec((FBLK, NM), lambda k: (k, 0)),
            pl.BlockSpec((1, B), lambda k: (0, 0)),
        ],
        out_specs=pl.BlockSpec((FBLK, B), lambda k: (k, 0)),
        out_shape=jax.ShapeDtypeStruct((FEAT, B), jnp.float32),
    )(wt, idx_row)


def kernel(x, weights):
    xt = jnp.transpose(x.reshape(B, FEAT))
    wt = jnp.transpose(weights.reshape(NM, FEAT))
    idx = _compute_indices(xt, wt)
    out_fm = _gather_rows(wt, idx.reshape(1, B))
    return jnp.transpose(out_fm).reshape(B, 64, 64, 3)


# speedup vs baseline: 3.0148x; 2.2874x over previous
import jax
import jax.numpy as jnp
from jax import lax
from jax.experimental import pallas as pl
from jax.experimental.pallas import tpu as pltpu

B = 256
NM = 1024
FEAT = 12288
KBLK = 1536
KSTEPS = FEAT // KBLK
FBLK = 1024
FSTEPS = FEAT // FBLK
GRID_N = 32


_JB = 128
_LB = 8


def _wt_transpose_kernel(wg_ref, out_ref):
    wb = wg_ref[...]
    pieces = []
    for l8 in range(_LB):
        for d in range(3):
            pieces.append(jnp.transpose(wb[:, d, l8, :]))
    out_ref[...] = jnp.concatenate(pieces, axis=0)


def _build_wt(wg):
    return pl.pallas_call(
        _wt_transpose_kernel,
        grid=(NM // _JB, 64 // _LB),
        in_specs=[pl.BlockSpec((_JB, 3, _LB, 64), lambda j, l: (j, 0, l, 0))],
        out_specs=pl.BlockSpec((3 * _LB * 64, _JB), lambda j, l: (l, j)),
        out_shape=jax.ShapeDtypeStruct((FEAT, NM), jnp.float32),
    )(wg)


def _argmin_kernel(xt_ref, wt_ref, idx_ref, acc_ref):
    k = pl.program_id(0)
    part = lax.dot_general(
        xt_ref[...], wt_ref[...],
        dimension_numbers=(((0,), (0,)), ((), ())),
        preferred_element_type=jnp.float32,
    )
    w2 = jnp.zeros((1, NM), jnp.float32)
    for c in range(KBLK // 128):
        blk = wt_ref[c * 128:(c + 1) * 128, :]
        w2 = w2 + jnp.sum(blk * blk, axis=0, keepdims=True)
    upd = w2 - 2.0 * part

    @pl.when(k == 0)
    def _init():
        acc_ref[...] = upd

    @pl.when(k > 0)
    def _acc():
        acc_ref[...] += upd

    @pl.when(k == KSTEPS - 1)
    def _finish():
        scores = acc_ref[...]
        minv = jnp.min(scores, axis=1, keepdims=True)
        iota = lax.broadcasted_iota(jnp.int32, scores.shape, 1)
        idx = jnp.min(jnp.where(scores == minv, iota, NM), axis=1,
                      keepdims=True)
        flat = (idx % GRID_N) * GRID_N + idx // GRID_N
        idx_ref[...] = flat


def _compute_indices(xt, wt):
    return pl.pallas_call(
        _argmin_kernel,
        grid=(KSTEPS,),
        in_specs=[
            pl.BlockSpec((KBLK, B), lambda k: (k, 0)),
            pl.BlockSpec((KBLK, NM), lambda k: (k, 0)),
        ],
        out_specs=pl.BlockSpec((B, 1), lambda k: (0, 0)),
        out_shape=jax.ShapeDtypeStruct((B, 1), jnp.int32),
        scratch_shapes=[pltpu.VMEM((B, NM), jnp.float32)],
    )(xt, wt)


def _gather_kernel(wt_ref, idx_ref, out_ref):
    onehot = jnp.where(
        lax.broadcasted_iota(jnp.int32, (NM, B), 0) == idx_ref[...],
        1.0, 0.0).astype(jnp.float32)
    out_ref[...] = lax.dot_general(
        wt_ref[...], onehot,
        dimension_numbers=(((1,), (0,)), ((), ())),
        preferred_element_type=jnp.float32,
    )


def _gather_rows(wt, idx_row):
    return pl.pallas_call(
        _gather_kernel,
        grid=(FSTEPS,),
        in_specs=[
            pl.BlockSpec((FBLK, NM), lambda k: (k, 0)),
            pl.BlockSpec((1, B), lambda k: (0, 0)),
        ],
        out_specs=pl.BlockSpec((FBLK, B), lambda k: (k, 0)),
        out_shape=jax.ShapeDtypeStruct((FEAT, B), jnp.float32),
    )(wt, idx_row)


def kernel(x, weights):
    xt = jnp.transpose(x, (1, 3, 2, 0)).reshape(FEAT, B)
    wg = jnp.transpose(weights, (0, 1, 4, 2, 3)).reshape(NM, 3, 64, 64)
    wt = _build_wt(wg)
    idx = _compute_indices(xt, wt)
    out_fm = _gather_rows(wt, idx.reshape(1, B))
    out4 = out_fm.reshape(64, 3, 64, B)
    return jnp.transpose(out4, (3, 0, 2, 1))


# speedup vs baseline: 3.1524x; 1.0456x over previous
import jax
import jax.numpy as jnp
from jax import lax
from jax.experimental import pallas as pl
from jax.experimental.pallas import tpu as pltpu

B = 256
NM = 1024
FEAT = 12288
KBLK = 1536
KSTEPS = FEAT // KBLK
GRID_N = 32
_LB = 8


def _fill_wt_scratch(wg_ref, wt_s):
    for l8 in range(_LB):
        for d in range(3):
            wt_s[(l8 * 3 + d) * 64:(l8 * 3 + d + 1) * 64, :] = (
                jnp.transpose(wg_ref[:, d, l8, :]))


def _argmin_kernel(xt_ref, wg_ref, idx_ref, acc_ref, wt_s):
    k = pl.program_id(0)
    _fill_wt_scratch(wg_ref, wt_s)
    part = lax.dot_general(
        xt_ref[...], wt_s[...],
        dimension_numbers=(((0,), (0,)), ((), ())),
        preferred_element_type=jnp.float32,
    )
    w2 = jnp.zeros((1, NM), jnp.float32)
    for c in range(KBLK // 128):
        blk = wt_s[c * 128:(c + 1) * 128, :]
        w2 = w2 + jnp.sum(blk * blk, axis=0, keepdims=True)
    upd = w2 - 2.0 * part

    @pl.when(k == 0)
    def _init():
        acc_ref[...] = upd

    @pl.when(k > 0)
    def _acc():
        acc_ref[...] += upd

    @pl.when(k == KSTEPS - 1)
    def _finish():
        scores = acc_ref[...]
        minv = jnp.min(scores, axis=1, keepdims=True)
        iota = lax.broadcasted_iota(jnp.int32, scores.shape, 1)
        idx = jnp.min(jnp.where(scores == minv, iota, NM), axis=1,
                      keepdims=True)
        flat = (idx % GRID_N) * GRID_N + idx // GRID_N
        idx_ref[...] = flat


def _compute_indices(xt, wg):
    return pl.pallas_call(
        _argmin_kernel,
        grid=(KSTEPS,),
        in_specs=[
            pl.BlockSpec((KBLK, B), lambda k: (k, 0)),
            pl.BlockSpec((NM, 3, _LB, 64), lambda k: (0, 0, k, 0)),
        ],
        out_specs=pl.BlockSpec((B, 1), lambda k: (0, 0)),
        out_shape=jax.ShapeDtypeStruct((B, 1), jnp.int32),
        scratch_shapes=[pltpu.VMEM((B, NM), jnp.float32),
                        pltpu.VMEM((KBLK, NM), jnp.float32)],
    )(xt, wg)


def _gather_kernel(wg_ref, idx_ref, out_ref, wt_s):
    _fill_wt_scratch(wg_ref, wt_s)
    onehot = jnp.where(
        lax.broadcasted_iota(jnp.int32, (NM, B), 0) == idx_ref[...],
        1.0, 0.0).astype(jnp.float32)
    out_ref[...] = lax.dot_general(
        wt_s[...], onehot,
        dimension_numbers=(((1,), (0,)), ((), ())),
        preferred_element_type=jnp.float32,
    )


def _gather_rows(wg, idx_row):
    return pl.pallas_call(
        _gather_kernel,
        grid=(KSTEPS,),
        in_specs=[
            pl.BlockSpec((NM, 3, _LB, 64), lambda k: (0, 0, k, 0)),
            pl.BlockSpec((1, B), lambda k: (0, 0)),
        ],
        out_specs=pl.BlockSpec((KBLK, B), lambda k: (k, 0)),
        out_shape=jax.ShapeDtypeStruct((FEAT, B), jnp.float32),
        scratch_shapes=[pltpu.VMEM((KBLK, NM), jnp.float32)],
    )(wg, idx_row)


def kernel(x, weights):
    xt = jnp.transpose(x, (1, 3, 2, 0)).reshape(FEAT, B)
    wg = jnp.transpose(weights, (0, 1, 4, 2, 3)).reshape(NM, 3, 64, 64)
    idx = _compute_indices(xt, wg)
    out_fm = _gather_rows(wg, idx.reshape(1, B))
    out4 = out_fm.reshape(64, 3, 64, B)
    return jnp.transpose(out4, (3, 0, 2, 1))
